# baseline (speedup 1.0000x reference)
import jax
import jax.numpy as jnp
from jax import lax
from jax.experimental import pallas as pl
from jax.experimental.pallas import tpu as pltpu
from jax.experimental.pallas import tpu_sc as plsc

ROWS = 8192
CH = 2048
L = 16
NC = 2
NS = 16
NW = NC * NS
ROWS_PER_W = ROWS // NW
R = 16
N_CHUNKS = ROWS_PER_W // R
N_JC = CH // L


def _body(x_hbm, perm_hbm, out_hbm, perm_v, in0_v, in1_v, out_v,
          sem_in0, sem_in1, sem_out):
    wid = lax.axis_index("s") * NC + lax.axis_index("c")
    base = wid * ROWS_PER_W
    ins = (in0_v, in1_v)
    sem_ins = (sem_in0, sem_in1)

    pltpu.sync_copy(perm_hbm, perm_v)

    def start_in(ci, b):
        pltpu.async_copy(x_hbm.at[pl.ds(base + ci * R, R)], ins[b],
                         sem_ins[b])

    def wait_in(b):
        pltpu.make_async_copy(x_hbm.at[pl.ds(base, R)], ins[b],
                              sem_ins[b]).wait()

    def start_out(ci):
        pltpu.async_copy(out_v, out_hbm.at[pl.ds(base + ci * R, R)], sem_out)

    def wait_out():
        pltpu.make_async_copy(out_v, out_hbm.at[pl.ds(base, R)],
                              sem_out).wait()

    start_in(0, 0)

    def chunk(p, carry):
        for b in range(2):
            ci = 2 * p + b
            wait_in(b)

            @pl.when(ci + 1 < N_CHUNKS)
            def _():
                start_in(ci + 1, 1 - b)

            @pl.when(ci >= 1)
            def _():
                wait_out()

            in_v = ins[b]

            @plsc.parallel_loop(0, N_JC, unroll=4)
            def _col(j):
                idx = perm_v[pl.ds(j * L, L)]
                for r in range(R):
                    rvec = jnp.full((L,), r, jnp.int32)
                    out_v[r, pl.ds(j * L, L)] = plsc.load_gather(
                        in_v, [rvec, idx])

            start_out(ci)
        return carry

    lax.fori_loop(0, N_CHUNKS // 2, chunk, 0)
    wait_out()


@jax.jit
def kernel(x, perm):
    f = pl.kernel(
        _body,
        out_type=jax.ShapeDtypeStruct((ROWS, CH), jnp.float32),
        mesh=plsc.VectorSubcoreMesh(core_axis_name="c", subcore_axis_name="s"),
        scratch_types=[
            pltpu.VMEM((CH,), jnp.int32),
            pltpu.VMEM((R, CH), jnp.float32),
            pltpu.VMEM((R, CH), jnp.float32),
            pltpu.VMEM((R, CH), jnp.float32),
            pltpu.SemaphoreType.DMA,
            pltpu.SemaphoreType.DMA,
            pltpu.SemaphoreType.DMA,
        ],
        compiler_params=pltpu.CompilerParams(needs_layout_passes=False),
    )
    return f(x, perm)
